# baseline (speedup 1.0000x reference)
import functools

import jax
import jax.numpy as jnp
from jax.experimental import pallas as pl


_LOG2E = 1.4426950408889634


def _project_kernel(x_ref, w_ref, a_ref, zb_ref, s_ref, t_ref):
    z = jnp.dot(x_ref[...], w_ref[...], preferred_element_type=jnp.float32)
    zb_ref[...] = z.astype(jnp.bfloat16)
    s_ref[...] = _LOG2E * jax.lax.dot_general(
        z, a_ref[0:1, :], (((1,), (1,)), ((), ())),
        preferred_element_type=jnp.float32)
    t_ref[...] = _LOG2E * jax.lax.dot_general(
        a_ref[1:2, :], z, (((1,), (1,)), ((), ())),
        preferred_element_type=jnp.float32)


def _half(s, t, zb, adj):
    x = s + t
    e = jnp.maximum(x, 0.2 * x)
    e = jnp.where(adj > 0, e, 0.0)
    m = jnp.max(e, axis=1, keepdims=True)
    p = jnp.exp2(e - m)
    l = jnp.sum(p, axis=1, keepdims=True)
    num = jnp.dot(p.astype(jnp.bfloat16), zb,
                  preferred_element_type=jnp.float32)
    return num / l


def _gat_kernel(br, nstream, s_ref, t_ref, zb_ref, *refs):
    adj_refs = refs[:nstream]
    o_ref = refs[nstream]
    t = t_ref[...]
    zb = zb_ref[...]
    for k in range(nstream):
        o_ref[k * br:(k + 1) * br, :] = _half(
            s_ref[k * br:(k + 1) * br, :], t, zb, adj_refs[k][...])


def _pick_block(n, target):
    for b in range(min(target, n), 0, -1):
        if n % b == 0:
            return b
    return n


def kernel(X, A, W, a):
    n, _ = X.shape
    dout = W.shape[1]
    a2r = a.reshape(2, dout).astype(jnp.float32)

    zb, s, t = pl.pallas_call(
        _project_kernel,
        out_shape=[
            jax.ShapeDtypeStruct((n, dout), jnp.bfloat16),
            jax.ShapeDtypeStruct((n, 1), jnp.float32),
            jax.ShapeDtypeStruct((1, n), jnp.float32),
        ],
    )(X, W, a2r)

    nstream = 5 if n % 5 == 0 else 2
    br = _pick_block(n // nstream, 80)
    ni = n // (nstream * br)

    h = pl.pallas_call(
        functools.partial(_gat_kernel, br, nstream),
        grid=(ni,),
        in_specs=[
            pl.BlockSpec((nstream * br, 1), lambda i: (i, 0)),
            pl.BlockSpec((1, n), lambda i: (0, 0)),
            pl.BlockSpec((n, dout), lambda i: (0, 0)),
        ] + [
            pl.BlockSpec((br, n),
                         functools.partial(lambda k, i: (nstream * i + k, 0), k))
            for k in range(nstream)
        ],
        out_specs=pl.BlockSpec((nstream * br, dout), lambda i: (i, 0)),
        out_shape=jax.ShapeDtypeStruct((n, dout), jnp.float32),
    )(s, t, zb, *([A] * nstream))
    return h


# speedup vs baseline: 1.0037x; 1.0037x over previous
import functools

import jax
import jax.numpy as jnp
from jax.experimental import pallas as pl
from jax.experimental.pallas import tpu as pltpu


_LOG2E = 1.4426950408889634


def _project_kernel(x_ref, w_ref, a_ref, zb_ref, s_ref, t_ref):
    z = jnp.dot(x_ref[...], w_ref[...], preferred_element_type=jnp.float32)
    zb_ref[...] = z.astype(jnp.bfloat16)
    s_ref[...] = _LOG2E * jax.lax.dot_general(
        z, a_ref[0:1, :], (((1,), (1,)), ((), ())),
        preferred_element_type=jnp.float32)
    t_ref[...] = _LOG2E * jax.lax.dot_general(
        a_ref[1:2, :], z, (((1,), (1,)), ((), ())),
        preferred_element_type=jnp.float32)


def _rows(s, t, zb, adj):
    x = s + t
    e = jnp.maximum(x, 0.2 * x)
    e = jnp.where(adj > 0, e, 0.0)
    m = jnp.max(e, axis=1, keepdims=True)
    p = jnp.exp2(e - m)
    l = jnp.sum(p, axis=1, keepdims=True)
    num = jnp.dot(p.astype(jnp.bfloat16), zb,
                  preferred_element_type=jnp.float32)
    return num / l


def _gat_kernel(br, ni, nbuf, s_ref, t_ref, zb_ref, a_hbm, o_ref, buf, sems):
    t = t_ref[...]
    zb = zb_ref[...]

    def copy_in(i, slot):
        return pltpu.make_async_copy(
            a_hbm.at[pl.ds(i * br, br), :], buf.at[slot], sems.at[slot])

    for k in range(nbuf - 1):
        copy_in(k, k).start()

    def step(i, carry):
        slot = jax.lax.rem(i, nbuf)
        nxt = i + nbuf - 1

        @pl.when(nxt < ni)
        def _prefetch():
            copy_in(nxt, jax.lax.rem(nxt, nbuf)).start()

        copy_in(i, slot).wait()
        o_ref[pl.ds(i * br, br), :] = _rows(
            s_ref[pl.ds(i * br, br), :], t, zb, buf[slot])
        return carry

    jax.lax.fori_loop(0, ni, step, 0)


def _pick_block(n, target):
    for b in range(min(target, n), 0, -1):
        if n % b == 0:
            return b
    return n


def kernel(X, A, W, a):
    n, _ = X.shape
    dout = W.shape[1]
    a2r = a.reshape(2, dout).astype(jnp.float32)

    zb, s, t = pl.pallas_call(
        _project_kernel,
        out_shape=[
            jax.ShapeDtypeStruct((n, dout), jnp.bfloat16),
            jax.ShapeDtypeStruct((n, 1), jnp.float32),
            jax.ShapeDtypeStruct((1, n), jnp.float32),
        ],
    )(X, W, a2r)

    br = _pick_block(n, 200)
    ni = n // br
    nbuf = min(3, ni)

    h = pl.pallas_call(
        functools.partial(_gat_kernel, br, ni, nbuf),
        in_specs=[
            pl.BlockSpec((n, 1), lambda: (0, 0)),
            pl.BlockSpec((1, n), lambda: (0, 0)),
            pl.BlockSpec((n, dout), lambda: (0, 0)),
            pl.BlockSpec(memory_space=pl.ANY),
        ],
        out_specs=pl.BlockSpec((n, dout), lambda: (0, 0)),
        out_shape=jax.ShapeDtypeStruct((n, dout), jnp.float32),
        scratch_shapes=[
            pltpu.VMEM((nbuf, br, n), jnp.float32),
            pltpu.SemaphoreType.DMA((nbuf,)),
        ],
    )(s, t, zb, A)
    return h


# speedup vs baseline: 1.0823x; 1.0782x over previous
import functools

import jax
import jax.numpy as jnp
from jax.experimental import pallas as pl
from jax.experimental.pallas import tpu as pltpu


_LOG2E = 1.4426950408889634


def _project_kernel(x_ref, w_ref, a_ref, zb_ref, s_ref, t_ref):
    z = jnp.dot(x_ref[...], w_ref[...], preferred_element_type=jnp.float32)
    zb_ref[...] = z.astype(jnp.bfloat16)
    s_ref[...] = _LOG2E * jax.lax.dot_general(
        z, a_ref[0:1, :], (((1,), (1,)), ((), ())),
        preferred_element_type=jnp.float32)
    t_ref[...] = _LOG2E * jax.lax.dot_general(
        a_ref[1:2, :], z, (((1,), (1,)), ((), ())),
        preferred_element_type=jnp.float32)


def _rows(s, t, zb, adj):
    x = s + t
    e = jnp.maximum(x, 0.2 * x)
    e = jnp.where(adj > 0, e, 0.0)
    m = jnp.max(e, axis=1, keepdims=True)
    p = jnp.exp2(e - m)
    l = jnp.sum(p, axis=1, keepdims=True)
    num = jnp.dot(p.astype(jnp.bfloat16), zb,
                  preferred_element_type=jnp.float32)
    return num / l


def _gat_kernel(br, s_ref, t_ref, zb_ref, adj0_ref, adj1_ref, o_ref):
    t = t_ref[...]
    zb = zb_ref[...]
    o_ref[0:br, :] = _rows(s_ref[0:br, :], t, zb, adj0_ref[...])
    o_ref[br:2 * br, :] = _rows(s_ref[br:2 * br, :], t, zb, adj1_ref[...])


def _pick_block(n, target):
    for b in range(min(target, n), 0, -1):
        if n % b == 0:
            return b
    return n


def kernel(X, A, W, a):
    n, _ = X.shape
    dout = W.shape[1]
    a2r = a.reshape(2, dout).astype(jnp.float32)

    zb, s, t = pl.pallas_call(
        _project_kernel,
        out_shape=[
            jax.ShapeDtypeStruct((n, dout), jnp.bfloat16),
            jax.ShapeDtypeStruct((n, 1), jnp.float32),
            jax.ShapeDtypeStruct((1, n), jnp.float32),
        ],
    )(X, W, a2r)

    nstream = 2
    br = _pick_block(n // nstream, 200)
    ni = n // (nstream * br)

    h = pl.pallas_call(
        functools.partial(_gat_kernel, br),
        grid=(ni,),
        in_specs=[
            pl.BlockSpec((nstream * br, 1), lambda i: (i, 0)),
            pl.BlockSpec((1, n), lambda i: (0, 0)),
            pl.BlockSpec((n, dout), lambda i: (0, 0)),
            pl.BlockSpec((br, n), lambda i: (2 * i, 0)),
            pl.BlockSpec((br, n), lambda i: (2 * i + 1, 0)),
        ],
        out_specs=pl.BlockSpec((nstream * br, dout), lambda i: (i, 0)),
        out_shape=jax.ShapeDtypeStruct((n, dout), jnp.float32),
    )(s, t, zb, A, A)
    return h


# speedup vs baseline: 1.0923x; 1.0093x over previous
import functools

import jax
import jax.numpy as jnp
from jax.experimental import pallas as pl
from jax.experimental.pallas import tpu as pltpu


_LOG2E = 1.4426950408889634


def _project_kernel(x_ref, w_ref, a_ref, zb_ref, s_ref, t_ref):
    z = jnp.dot(x_ref[...], w_ref[...], preferred_element_type=jnp.float32)
    zb_ref[...] = z.astype(jnp.bfloat16)
    s_ref[...] = _LOG2E * jax.lax.dot_general(
        z, a_ref[0:1, :], (((1,), (1,)), ((), ())),
        preferred_element_type=jnp.float32)
    t_ref[...] = _LOG2E * jax.lax.dot_general(
        a_ref[1:2, :], z, (((1,), (1,)), ((), ())),
        preferred_element_type=jnp.float32)


def _rows(s, t, zb, adj):
    x = s + t
    e = jnp.maximum(x, 0.2 * x)
    e = jnp.where(adj > 0, e, 0.0)
    m = jnp.max(e, axis=1, keepdims=True)
    p = jnp.exp2(e - m)
    l = jnp.sum(p, axis=1, keepdims=True)
    num = jnp.dot(p.astype(jnp.bfloat16), zb,
                  preferred_element_type=jnp.float32)
    return num / l


def _gat_kernel(s0_ref, s1_ref, t_ref, zb_ref, adj0_ref, adj1_ref, o_ref):
    t = t_ref[...]
    zb = zb_ref[...]
    o_ref[0] = _rows(s0_ref[...], t, zb, adj0_ref[...])
    o_ref[1] = _rows(s1_ref[...], t, zb, adj1_ref[...])


def _pick_block(n, target):
    for b in range(min(target, n), 0, -1):
        if n % b == 0:
            return b
    return n


def kernel(X, A, W, a):
    n, _ = X.shape
    dout = W.shape[1]
    a2r = a.reshape(2, dout).astype(jnp.float32)

    zb, s, t = pl.pallas_call(
        _project_kernel,
        out_shape=[
            jax.ShapeDtypeStruct((n, dout), jnp.bfloat16),
            jax.ShapeDtypeStruct((n, 1), jnp.float32),
            jax.ShapeDtypeStruct((1, n), jnp.float32),
        ],
    )(X, W, a2r)

    br = _pick_block(n // 2, 200)
    ni = n // (2 * br)

    h2 = pl.pallas_call(
        _gat_kernel,
        grid=(ni,),
        in_specs=[
            pl.BlockSpec((br, 1), lambda i: (i, 0)),
            pl.BlockSpec((br, 1), functools.partial(
                lambda ni_, i: (i + ni_, 0), ni)),
            pl.BlockSpec((1, n), lambda i: (0, 0)),
            pl.BlockSpec((n, dout), lambda i: (0, 0)),
            pl.BlockSpec((br, n), lambda i: (i, 0)),
            pl.BlockSpec((br, n), functools.partial(
                lambda ni_, i: (i + ni_, 0), ni)),
        ],
        out_specs=pl.BlockSpec((2, br, dout), lambda i: (0, i, 0)),
        out_shape=jax.ShapeDtypeStruct((2, n // 2, dout), jnp.float32),
    )(s, s, t, zb, A, A)
    return h2.reshape(n, dout)
